# baseline (speedup 1.0000x reference)
import functools

import jax
import jax.numpy as jnp
from jax import lax
from jax.experimental import pallas as pl
from jax.experimental.pallas import tpu as pltpu
from jax.experimental.pallas import tpu_sc as plsc

EMBED_DIM = 128
EPS = 1e-06

TURN_OFF = 1
CASTLE_OFF = 3
EP_OFF = 19
META_ROWS = 88


def _table_body(piece_ref, square_ref, turn_ref, castle_ref, ep_ref, w_ref,
                piecew_ref, squaretw_ref, scale_ref, meta_ref):
    w = w_ref[...]
    piece = piece_ref[...]
    square = square_ref[...]
    piecew = piece * w
    piecew_ref[...] = jnp.concatenate(
        [piecew, jnp.zeros((3, EMBED_DIM), jnp.float32)], axis=0)
    squaretw_ref[...] = square * w
    comb = square[:, None, :] + piece[None, :, :]
    ms = jnp.mean(comb * comb, axis=-1)
    scale = lax.rsqrt(ms + EPS)
    scale_ref[...] = jnp.concatenate(
        [jnp.concatenate([scale, jnp.zeros((64, 3), jnp.float32)], axis=1),
         jnp.zeros((1, 16), jnp.float32)], axis=0)
    rows = jnp.concatenate(
        [jnp.zeros((1, EMBED_DIM), jnp.float32), turn_ref[...], castle_ref[...],
         ep_ref[...], jnp.zeros((META_ROWS - 84, EMBED_DIM), jnp.float32)],
        axis=0)
    mms = jnp.mean(rows * rows, axis=1, keepdims=True)
    meta_ref[...] = rows * lax.rsqrt(mms + EPS) * w


def _prep_tables(piece, square, turn, castle, ep, w):
    return pl.pallas_call(
        _table_body,
        out_shape=(
            jax.ShapeDtypeStruct((16, EMBED_DIM), jnp.float32),
            jax.ShapeDtypeStruct((64, EMBED_DIM), jnp.float32),
            jax.ShapeDtypeStruct((65, 16), jnp.float32),
            jax.ShapeDtypeStruct((META_ROWS, EMBED_DIM), jnp.float32),
        ),
    )(piece, square, turn, castle, ep, w.reshape(1, EMBED_DIM))


def _idx_body(board_ref, meta_ref, out_ref):
    board = board_ref[...]
    m = meta_ref[...]
    zeros1 = jnp.zeros((board.shape[0], 1), jnp.int32)
    pad = jnp.zeros((board.shape[0], 4), jnp.int32)
    out_ref[...] = jnp.concatenate(
        [zeros1, board,
         m[:, 0:1] + TURN_OFF, m[:, 1:2] + CASTLE_OFF, m[:, 2:3] + EP_OFF,
         pad],
        axis=1)


def _prep_idx(board, meta):
    b = board.shape[0]
    blk = 2048
    assert b % blk == 0
    return pl.pallas_call(
        _idx_body,
        grid=(b // blk,),
        in_specs=[pl.BlockSpec((blk, 64), lambda i: (i, 0)),
                  pl.BlockSpec((blk, 3), lambda i: (i, 0))],
        out_specs=pl.BlockSpec((blk, 72), lambda i: (i, 0)),
        out_shape=jax.ShapeDtypeStruct((b, 72), jnp.int32),
    )(board, meta)


def _sc_encode(piecew, squaretw, scale, metan, idx2d):
    info = plsc.get_sparse_core_info()
    nc, ns = info.num_cores, info.num_subcores
    nw = nc * ns
    bsz, kw = idx2d.shape
    tok = 68
    row_e = tok * EMBED_DIM
    grp = 4
    blk_e = grp * row_e
    half = bsz // nw // 2
    ngrp = half // grp
    assert bsz % (2 * nw * grp) == 0 and ngrp % 2 == 0

    mesh = plsc.VectorSubcoreMesh(core_axis_name="c", subcore_axis_name="s")

    @functools.partial(
        pl.kernel,
        out_type=jax.ShapeDtypeStruct((bsz * row_e,), jnp.float32),
        mesh=mesh,
        scratch_types=[
            pltpu.VMEM((half, kw), jnp.int32),
            pltpu.VMEM((16 * EMBED_DIM,), jnp.float32),
            pltpu.VMEM((64 * EMBED_DIM,), jnp.float32),
            pltpu.VMEM((65 * 16,), jnp.float32),
            pltpu.VMEM((META_ROWS * EMBED_DIM,), jnp.float32),
            [pltpu.VMEM((blk_e,), jnp.float32)] * 2,
            [pltpu.SemaphoreType.DMA] * 2,
        ],
    )
    def encode_kernel(pc_hbm, sq_hbm, sc_hbm, mn_hbm, idx_hbm, out_hbm,
                      idx_v, pc_v, sq_v, sc_v, mn_v, stage, ssem):
        wid = lax.axis_index("s") * nc + lax.axis_index("c")
        pltpu.sync_copy(pc_hbm, pc_v)
        pltpu.sync_copy(sq_hbm, sq_v)
        pltpu.sync_copy(sc_hbm, sc_v)
        pltpu.sync_copy(mn_hbm, mn_v)

        fzero = jnp.zeros((16,), jnp.float32)

        def compute(g, b):
            j0 = g * grp

            def jloop(jj, carry):
                j = j0 + jj
                soff = jj * row_e

                def tloop(t, carry2):
                    sv = idx_v[j, pl.ds(1 + 16 * t, 16)]
                    s0 = 16 * t
                    for k in range(16):
                        p = sv[k]
                        s = s0 + k
                        sc = sc_v[pl.ds(s * 16 + p, 16)][0]
                        pb = p * EMBED_DIM
                        sb = s * EMBED_DIM
                        to = soff + (1 + s) * EMBED_DIM
                        for q in range(8):
                            x = (pc_v[pl.ds(pb + 16 * q, 16)] +
                                 sq_v[pl.ds(sb + 16 * q, 16)]) * sc
                            stage[b][pl.ds(to + 16 * q, 16)] = x
                    return carry2

                lax.fori_loop(0, 4, tloop, 0)

                for q in range(8):
                    stage[b][pl.ds(soff + 16 * q, 16)] = fzero
                mrow = idx_v[j, pl.ds(56, 16)]
                for t in range(3):
                    mb2 = mrow[9 + t] * EMBED_DIM
                    to = soff + (65 + t) * EMBED_DIM
                    for q in range(8):
                        stage[b][pl.ds(to + 16 * q, 16)] = (
                            mn_v[pl.ds(mb2 + 16 * q, 16)])
                return carry

            lax.fori_loop(0, grp, jloop, 0)

        def store(ebase, g, b):
            pltpu.async_copy(stage[b], out_hbm.at[pl.ds(ebase + g * blk_e,
                                                        blk_e)], ssem[b])

        def store_wait(ebase, b):
            pltpu.make_async_copy(stage[b], out_hbm.at[pl.ds(ebase, blk_e)],
                                  ssem[b]).wait()

        def hloop(h, carry):
            bbase = (wid * 2 + h) * half
            ebase = bbase * row_e
            pltpu.sync_copy(idx_hbm.at[pl.ds(bbase, half)], idx_v)

            def body(i, carry2):
                for b in range(2):
                    @pl.when(i > 0)
                    def _():
                        store_wait(ebase, b)
                    compute(i * 2 + b, b)
                    store(ebase, i * 2 + b, b)
                return carry2

            lax.fori_loop(0, ngrp // 2, body, 0)

            for b in range(2):
                store_wait(ebase, b)
            return carry

        lax.fori_loop(0, 2, hloop, 0)

    return encode_kernel(piecew.reshape(-1), squaretw.reshape(-1),
                         scale.reshape(-1), metan.reshape(-1), idx2d)


def kernel(board_tensor, metadata, piece_table, square_table, turn_table,
           castling_table, en_passant_table, rms_weight):
    b = board_tensor.shape[0]
    board = board_tensor.astype(jnp.int32)
    meta = metadata.astype(jnp.int32)

    piecew, squaretw, scale, metan = _prep_tables(
        piece_table, square_table, turn_table, castling_table,
        en_passant_table, rms_weight)
    idx = _prep_idx(board, meta)
    out = _sc_encode(piecew, squaretw, scale, metan, idx)
    return out.reshape(b, 68, EMBED_DIM)


# speedup vs baseline: 1.1052x; 1.1052x over previous
import functools

import jax
import jax.numpy as jnp
from jax import lax
from jax.experimental import pallas as pl
from jax.experimental.pallas import tpu as pltpu
from jax.experimental.pallas import tpu_sc as plsc

EMBED_DIM = 128
EPS = 1e-06

TURN_OFF = 832
CASTLE_OFF = TURN_OFF + 2
EP_OFF = CASTLE_OFF + 16
ZERO_ROW = EP_OFF + 65
TABLE_ROWS = 928


def _table_body(piece_ref, square_ref, turn_ref, castle_ref, ep_ref, w_ref, out_ref):
    piece = piece_ref[...]
    square = square_ref[...]
    comb = (square[:, None, :] + piece[None, :, :]).reshape(832, EMBED_DIM)
    zeros = jnp.zeros((TABLE_ROWS - ZERO_ROW, EMBED_DIM), jnp.float32)
    rows = jnp.concatenate(
        [comb, turn_ref[...], castle_ref[...], ep_ref[...], zeros], axis=0)
    ms = jnp.mean(rows * rows, axis=1, keepdims=True)
    out_ref[...] = rows * lax.rsqrt(ms + EPS) * w_ref[...]


def _prep_table(piece, square, turn, castle, ep, w):
    return pl.pallas_call(
        _table_body,
        out_shape=jax.ShapeDtypeStruct((TABLE_ROWS, EMBED_DIM), jnp.float32),
    )(piece, square, turn, castle, ep, w.reshape(1, EMBED_DIM))


def _idx_body(board_ref, meta_ref, out_ref):
    board = board_ref[...]
    offs = lax.broadcasted_iota(jnp.int32, (1, 64), 1) * 13
    m = meta_ref[...]
    cls = jnp.full((board.shape[0], 1), ZERO_ROW, jnp.int32)
    out_ref[...] = jnp.concatenate(
        [cls, board + offs,
         m[:, 0:1] + TURN_OFF, m[:, 1:2] + CASTLE_OFF, m[:, 2:3] + EP_OFF],
        axis=1)


def _prep_idx(board, meta):
    b = board.shape[0]
    blk = 2048
    assert b % blk == 0
    return pl.pallas_call(
        _idx_body,
        grid=(b // blk,),
        in_specs=[pl.BlockSpec((blk, 64), lambda i: (i, 0)),
                  pl.BlockSpec((blk, 3), lambda i: (i, 0))],
        out_specs=pl.BlockSpec((blk, 68), lambda i: (i, 0)),
        out_shape=jax.ShapeDtypeStruct((b, 68), jnp.int32),
    )(board, meta)


def _sc_gather(table, idx2d, total_rows):
    info = plsc.get_sparse_core_info()
    nc, ns = info.num_cores, info.num_subcores
    nw = nc * ns
    k = 128
    chunks_total = idx2d.shape[0]
    assert chunks_total % nw == 0
    chunks = chunks_total // nw
    per_w = chunks * k

    mesh = plsc.VectorSubcoreMesh(core_axis_name="c", subcore_axis_name="s")

    @functools.partial(
        pl.kernel,
        out_type=jax.ShapeDtypeStruct((total_rows, EMBED_DIM), jnp.float32),
        mesh=mesh,
        scratch_types=[
            pltpu.VMEM((chunks, k), jnp.int32),
            pltpu.VMEM((k, EMBED_DIM), jnp.float32),
            pltpu.SemaphoreType.DMA,
        ],
    )
    def gather_kernel(table_hbm, idx_hbm, out_hbm, idx_v, rows_v, sem):
        wid = lax.axis_index("s") * nc + lax.axis_index("c")
        pltpu.sync_copy(idx_hbm.at[pl.ds(wid * chunks, chunks)], idx_v)
        base = wid * per_w

        def step(j, carry):
            pltpu.async_copy(table_hbm.at[idx_v.at[j]], rows_v, sem).wait()
            pltpu.sync_copy(rows_v, out_hbm.at[pl.ds(base + j * k, k)])
            return carry

        lax.fori_loop(0, chunks, step, 0)

    return gather_kernel(table, idx2d)


def kernel(board_tensor, metadata, piece_table, square_table, turn_table,
           castling_table, en_passant_table, rms_weight):
    b = board_tensor.shape[0]
    board = board_tensor.astype(jnp.int32)
    meta = metadata.astype(jnp.int32)

    table = _prep_table(piece_table, square_table, turn_table,
                        castling_table, en_passant_table, rms_weight)
    idx = _prep_idx(board, meta)
    total_rows = b * 68
    idx2d = idx.reshape(total_rows // 128, 128)
    out = _sc_gather(table, idx2d, total_rows)
    return out.reshape(b, 68, EMBED_DIM)


# speedup vs baseline: 1.6827x; 1.5225x over previous
import functools

import jax
import jax.numpy as jnp
from jax import lax
from jax.experimental import pallas as pl
from jax.experimental.pallas import tpu as pltpu
from jax.experimental.pallas import tpu_sc as plsc

EMBED_DIM = 128
EPS = 1e-06

TURN_OFF = 832
CASTLE_OFF = TURN_OFF + 2
EP_OFF = CASTLE_OFF + 16
ZERO_ROW = EP_OFF + 65
TABLE_ROWS = 928


def _table_body(piece_ref, square_ref, turn_ref, castle_ref, ep_ref, w_ref, out_ref):
    piece = piece_ref[...]
    square = square_ref[...]
    comb = (square[:, None, :] + piece[None, :, :]).reshape(832, EMBED_DIM)
    zeros = jnp.zeros((TABLE_ROWS - ZERO_ROW, EMBED_DIM), jnp.float32)
    rows = jnp.concatenate(
        [comb, turn_ref[...], castle_ref[...], ep_ref[...], zeros], axis=0)
    ms = jnp.mean(rows * rows, axis=1, keepdims=True)
    out_ref[...] = rows * lax.rsqrt(ms + EPS) * w_ref[...]


def _prep_table(piece, square, turn, castle, ep, w, copies):
    full = lambda i: (0, 0)
    return pl.pallas_call(
        _table_body,
        grid=(copies,),
        in_specs=[pl.BlockSpec((13, EMBED_DIM), full),
                  pl.BlockSpec((64, EMBED_DIM), full),
                  pl.BlockSpec((2, EMBED_DIM), full),
                  pl.BlockSpec((16, EMBED_DIM), full),
                  pl.BlockSpec((65, EMBED_DIM), full),
                  pl.BlockSpec((1, EMBED_DIM), full)],
        out_specs=pl.BlockSpec((TABLE_ROWS, EMBED_DIM), lambda i: (i, 0)),
        out_shape=jax.ShapeDtypeStruct((copies * TABLE_ROWS, EMBED_DIM),
                                       jnp.float32),
    )(piece, square, turn, castle, ep, w.reshape(1, EMBED_DIM))


def _idx_body(bw, board_ref, meta_ref, out_ref):
    board = board_ref[...]
    blk = board.shape[0]
    offs = lax.broadcasted_iota(jnp.int32, (1, 64), 1) * 13
    m = meta_ref[...]
    cls = jnp.full((blk, 1), ZERO_ROW, jnp.int32)
    pid = pl.program_id(0)
    ro = (lax.broadcasted_iota(jnp.int32, (blk, 1), 0) // bw
          + pid * (blk // bw)) * TABLE_ROWS
    out_ref[...] = jnp.concatenate(
        [cls, board + offs,
         m[:, 0:1] + TURN_OFF, m[:, 1:2] + CASTLE_OFF, m[:, 2:3] + EP_OFF],
        axis=1) + ro


def _prep_idx(board, meta, bw):
    b = board.shape[0]
    blk = 2048
    assert b % blk == 0 and blk % bw == 0
    return pl.pallas_call(
        functools.partial(_idx_body, bw),
        grid=(b // blk,),
        in_specs=[pl.BlockSpec((blk, 64), lambda i: (i, 0)),
                  pl.BlockSpec((blk, 3), lambda i: (i, 0))],
        out_specs=pl.BlockSpec((blk, 68), lambda i: (i, 0)),
        out_shape=jax.ShapeDtypeStruct((b, 68), jnp.int32),
    )(board, meta)


def _sc_gather(table, idx2d, total_rows):
    info = plsc.get_sparse_core_info()
    nc, ns = info.num_cores, info.num_subcores
    nw = nc * ns
    k = 128
    chunks_total = idx2d.shape[0]
    assert chunks_total % nw == 0
    chunks = chunks_total // nw
    per_w = chunks * k

    mesh = plsc.VectorSubcoreMesh(core_axis_name="c", subcore_axis_name="s")

    @functools.partial(
        pl.kernel,
        out_type=jax.ShapeDtypeStruct((total_rows, EMBED_DIM), jnp.float32),
        mesh=mesh,
        scratch_types=[
            pltpu.VMEM((chunks, k), jnp.int32),
            pltpu.VMEM((k, EMBED_DIM), jnp.float32),
            pltpu.SemaphoreType.DMA,
        ],
    )
    def gather_kernel(table_hbm, idx_hbm, out_hbm, idx_v, rows_v, sem):
        wid = lax.axis_index("s") * nc + lax.axis_index("c")
        pltpu.sync_copy(idx_hbm.at[pl.ds(wid * chunks, chunks)], idx_v)
        base = wid * per_w

        def step(j, carry):
            pltpu.async_copy(table_hbm.at[idx_v.at[j]], rows_v, sem).wait()
            pltpu.sync_copy(rows_v, out_hbm.at[pl.ds(base + j * k, k)])
            return carry

        lax.fori_loop(0, chunks, step, 0)

    return gather_kernel(table, idx2d)


def kernel(board_tensor, metadata, piece_table, square_table, turn_table,
           castling_table, en_passant_table, rms_weight):
    b = board_tensor.shape[0]
    board = board_tensor.astype(jnp.int32)
    meta = metadata.astype(jnp.int32)

    info = plsc.get_sparse_core_info()
    nw = info.num_cores * info.num_subcores
    table = _prep_table(piece_table, square_table, turn_table,
                        castling_table, en_passant_table, rms_weight, nw)
    idx = _prep_idx(board, meta, b // nw)
    total_rows = b * 68
    idx2d = idx.reshape(total_rows // 128, 128)
    out = _sc_gather(table, idx2d, total_rows)
    return out.reshape(b, 68, EMBED_DIM)


# speedup vs baseline: 1.8877x; 1.1218x over previous
import functools

import jax
import jax.numpy as jnp
from jax import lax
from jax.experimental import pallas as pl
from jax.experimental.pallas import tpu as pltpu
from jax.experimental.pallas import tpu_sc as plsc

EMBED_DIM = 128
EPS = 1e-06

TURN_OFF = 832
CASTLE_OFF = TURN_OFF + 2
EP_OFF = CASTLE_OFF + 16
ZERO_ROW = EP_OFF + 65
TABLE_ROWS = 928


def _table_body(piece_ref, square_ref, turn_ref, castle_ref, ep_ref, w_ref, out_ref):
    piece = piece_ref[...]
    square = square_ref[...]
    comb = (square[:, None, :] + piece[None, :, :]).reshape(832, EMBED_DIM)
    zeros = jnp.zeros((TABLE_ROWS - ZERO_ROW, EMBED_DIM), jnp.float32)
    rows = jnp.concatenate(
        [comb, turn_ref[...], castle_ref[...], ep_ref[...], zeros], axis=0)
    ms = jnp.mean(rows * rows, axis=1, keepdims=True)
    out_ref[...] = rows * lax.rsqrt(ms + EPS) * w_ref[...]


def _prep_table(piece, square, turn, castle, ep, w, copies):
    full = lambda i: (0, 0)
    return pl.pallas_call(
        _table_body,
        grid=(copies,),
        in_specs=[pl.BlockSpec((13, EMBED_DIM), full),
                  pl.BlockSpec((64, EMBED_DIM), full),
                  pl.BlockSpec((2, EMBED_DIM), full),
                  pl.BlockSpec((16, EMBED_DIM), full),
                  pl.BlockSpec((65, EMBED_DIM), full),
                  pl.BlockSpec((1, EMBED_DIM), full)],
        out_specs=pl.BlockSpec((TABLE_ROWS, EMBED_DIM), lambda i: (i, 0)),
        out_shape=jax.ShapeDtypeStruct((copies * TABLE_ROWS, EMBED_DIM),
                                       jnp.float32),
    )(piece, square, turn, castle, ep, w.reshape(1, EMBED_DIM))


def _idx_body(bw, board_ref, meta_ref, out_ref):
    board = board_ref[...]
    blk = board.shape[0]
    offs = lax.broadcasted_iota(jnp.int32, (1, 64), 1) * 13
    m = meta_ref[...]
    cls = jnp.full((blk, 1), ZERO_ROW, jnp.int32)
    pid = pl.program_id(0)
    ro = (lax.broadcasted_iota(jnp.int32, (blk, 1), 0) // bw
          + pid * (blk // bw)) * TABLE_ROWS
    out_ref[...] = jnp.concatenate(
        [cls, board + offs,
         m[:, 0:1] + TURN_OFF, m[:, 1:2] + CASTLE_OFF, m[:, 2:3] + EP_OFF],
        axis=1) + ro


def _prep_idx(board, meta, bw):
    b = board.shape[0]
    blk = 2048
    assert b % blk == 0 and blk % bw == 0
    return pl.pallas_call(
        functools.partial(_idx_body, bw),
        grid=(b // blk,),
        in_specs=[pl.BlockSpec((blk, 64), lambda i: (i, 0)),
                  pl.BlockSpec((blk, 3), lambda i: (i, 0))],
        out_specs=pl.BlockSpec((blk, 68), lambda i: (i, 0)),
        out_shape=jax.ShapeDtypeStruct((b, 68), jnp.int32),
    )(board, meta)


def _sc_gather(table, idx2d, total_rows):
    info = plsc.get_sparse_core_info()
    nc, ns = info.num_cores, info.num_subcores
    nw = nc * ns
    k = 128
    chunks_total = idx2d.shape[0]
    assert chunks_total % nw == 0
    chunks = chunks_total // nw
    per_w = chunks * k

    nbuf = 4
    assert chunks % nbuf == 0
    mesh = plsc.VectorSubcoreMesh(core_axis_name="c", subcore_axis_name="s")

    @functools.partial(
        pl.kernel,
        out_type=jax.ShapeDtypeStruct((total_rows, EMBED_DIM), jnp.float32),
        mesh=mesh,
        scratch_types=[
            pltpu.VMEM((chunks, k), jnp.int32),
            [pltpu.VMEM((k, EMBED_DIM), jnp.float32)] * nbuf,
            [pltpu.SemaphoreType.DMA] * nbuf,
            [pltpu.SemaphoreType.DMA] * nbuf,
        ],
    )
    def gather_kernel(table_hbm, idx_hbm, out_hbm, idx_v, rows, gsem, ssem):
        wid = lax.axis_index("s") * nc + lax.axis_index("c")
        pltpu.sync_copy(idx_hbm.at[pl.ds(wid * chunks, chunks)], idx_v)
        base = wid * per_w

        def gather(j, b):
            pltpu.async_copy(table_hbm.at[idx_v.at[j]], rows[b], gsem[b])

        def store(j, b):
            pltpu.async_copy(rows[b], out_hbm.at[pl.ds(base + j * k, k)],
                             ssem[b])

        def gather_wait(b):
            pltpu.make_async_copy(table_hbm.at[idx_v.at[0]], rows[b],
                                  gsem[b]).wait()

        def store_wait(b):
            pltpu.make_async_copy(rows[b], out_hbm.at[pl.ds(base, k)],
                                  ssem[b]).wait()

        for b in range(nbuf):
            gather(b, b)

        def body(i, carry):
            j0 = i * nbuf
            for b in range(nbuf):
                gather_wait(b)
                store(j0 + b, b)
            for b in range(nbuf):
                store_wait(b)
                gather(j0 + nbuf + b, b)
            return carry

        lax.fori_loop(0, chunks // nbuf - 1, body, 0)

        j0 = chunks - nbuf
        for b in range(nbuf):
            gather_wait(b)
            store(j0 + b, b)
        for b in range(nbuf):
            store_wait(b)

    return gather_kernel(table, idx2d)


def kernel(board_tensor, metadata, piece_table, square_table, turn_table,
           castling_table, en_passant_table, rms_weight):
    b = board_tensor.shape[0]
    board = board_tensor.astype(jnp.int32)
    meta = metadata.astype(jnp.int32)

    info = plsc.get_sparse_core_info()
    nw = info.num_cores * info.num_subcores
    table = _prep_table(piece_table, square_table, turn_table,
                        castling_table, en_passant_table, rms_weight, nw)
    idx = _prep_idx(board, meta, b // nw)
    total_rows = b * 68
    idx2d = idx.reshape(total_rows // 128, 128)
    out = _sc_gather(table, idx2d, total_rows)
    return out.reshape(b, 68, EMBED_DIM)
